# baseline (speedup 1.0000x reference)
import jax
import jax.numpy as jnp
from jax.experimental import pallas as pl

ROWS, COLS = 16384, 1024
BLOCK_ROWS = 512


def _body(mask_ref, data_ref, out_ref):
    f = 1.0 + mask_ref[...]
    out_ref[...] = data_ref[...] * f


def kernel(data, applied_mask):
    mask_f = applied_mask.astype(jnp.float32).reshape(ROWS, 1)
    grid = (ROWS // BLOCK_ROWS,)
    return pl.pallas_call(
        _body,
        grid=grid,
        in_specs=[
            pl.BlockSpec((BLOCK_ROWS, 1), lambda i: (i, 0)),
            pl.BlockSpec((BLOCK_ROWS, COLS), lambda i: (i, 0)),
        ],
        out_specs=pl.BlockSpec((BLOCK_ROWS, COLS), lambda i: (i, 0)),
        out_shape=jax.ShapeDtypeStruct((ROWS, COLS), jnp.float32),
    )(mask_f, data)
